# Initial kernel scaffold; baseline (speedup 1.0000x reference)
import jax
import jax.numpy as jnp
from jax.experimental import pallas as pl


def kernel(x, table):
    raise NotImplementedError("write your pallas kernel here")


# speedup vs baseline: 4.3020x; 4.3020x over previous
import functools

import jax
import jax.numpy as jnp
from jax import lax
from jax.experimental import pallas as pl
from jax.experimental.pallas import tpu as pltpu
from jax.experimental.pallas import tpu_sc as plsc

NUM_BUCKETS = 1000000
EMB_DIM = 32

_ROWS = 16384
_COLS = 200
_TOTAL = _ROWS * _COLS
_W = 128
_NROW = _TOTAL // _W
_CHUNK = 8

_info = plsc.get_sparse_core_info()
_NC, _NS = _info.num_cores, _info.num_subcores
_NW = _NC * _NS
_ROWS_PER_W = _NROW // _NW
_CHUNKS_PER_W = _ROWS_PER_W // _CHUNK


def _sc_body(x_hbm, table_hbm, out_hbm, idx_v, rows_v, sem):
    wid = lax.axis_index("s") * _NC + lax.axis_index("c")
    base = wid * _ROWS_PER_W

    def chunk(g, carry):
        row = base + g * _CHUNK
        pltpu.sync_copy(x_hbm.at[pl.ds(row, _CHUNK)], idx_v)

        def hash_body(i, c):
            off = i * 16
            for j in range(_CHUNK):
                v = idx_v[j, pl.ds(off, 16)]
                h = jnp.where(v == 0, 0, v % (NUM_BUCKETS - 1) + 1)
                idx_v[j, pl.ds(off, 16)] = h
            return c

        lax.fori_loop(0, _W // 16, hash_body, 0)

        copies = [
            pltpu.async_copy(table_hbm.at[idx_v.at[j]], rows_v.at[j], sem)
            for j in range(_CHUNK)
        ]
        for c in copies:
            c.wait()

        pltpu.sync_copy(rows_v, out_hbm.at[pl.ds(row, _CHUNK)])
        return carry

    lax.fori_loop(0, _CHUNKS_PER_W, chunk, 0)


@jax.jit
def kernel(x, table):
    x2 = x.reshape(_NROW, _W)
    run = functools.partial(
        pl.kernel,
        mesh=plsc.VectorSubcoreMesh(core_axis_name="c", subcore_axis_name="s"),
        out_type=jax.ShapeDtypeStruct((_NROW, _W, EMB_DIM), jnp.float32),
        scratch_types=[
            pltpu.VMEM((_CHUNK, _W), jnp.int32),
            pltpu.VMEM((_CHUNK, _W, EMB_DIM), jnp.float32),
            pltpu.SemaphoreType.DMA,
        ],
        compiler_params=pltpu.CompilerParams(use_tc_tiling_on_sc=False),
    )(_sc_body)
    out = run(x2, table)
    return out.reshape(_ROWS, _COLS, EMB_DIM)


# speedup vs baseline: 4.9133x; 1.1421x over previous
import functools

import jax
import jax.numpy as jnp
import numpy as np
from jax import lax
from jax.experimental import pallas as pl
from jax.experimental.pallas import tpu as pltpu
from jax.experimental.pallas import tpu_sc as plsc

NUM_BUCKETS = 1000000
EMB_DIM = 32

_ROWS = 16384
_COLS = 200
_TOTAL = _ROWS * _COLS
_W = 128
_NROW = _TOTAL // _W
_CHUNK = 8

_info = plsc.get_sparse_core_info()
_NC, _NS = _info.num_cores, _info.num_subcores
_NW = _NC * _NS
_ROWS_PER_W = _NROW // _NW
_CHUNKS_PER_W = _ROWS_PER_W // _CHUNK

_D = NUM_BUCKETS - 1
_RECIP = np.float32(1.0 / _D)


def _hash16(v):
    q = (v.astype(jnp.float32) * _RECIP).astype(jnp.int32)
    r = v - q * _D
    r = jnp.where(r < 0, r + _D, r)
    r = jnp.where(r >= _D, r - _D, r)
    return jnp.where(v == 0, 0, r + 1)


def _sc_body(x_hbm, table_hbm, out_hbm, idx0, idx1, rows0, rows1, sem0, sem1):
    idx = (idx0, idx1)
    rows = (rows0, rows1)
    sem = (sem0, sem1)
    wid = lax.axis_index("s") * _NC + lax.axis_index("c")
    base = wid * _ROWS_PER_W

    def prep(chunk_i, b):
        row = base + chunk_i * _CHUNK
        pltpu.sync_copy(x_hbm.at[pl.ds(row, _CHUNK)], idx[b])

        def hash_body(i, c):
            off = i * 16
            for j in range(_CHUNK):
                idx[b][j, pl.ds(off, 16)] = _hash16(idx[b][j, pl.ds(off, 16)])
            return c

        lax.fori_loop(0, _W // 16, hash_body, 0)
        for j in range(_CHUNK):
            pltpu.async_copy(table_hbm.at[idx[b].at[j]], rows[b].at[j], sem[b])

    def finish(chunk_i, b):
        for j in range(_CHUNK):
            pltpu.make_async_copy(
                table_hbm.at[idx[b].at[j]], rows[b].at[j], sem[b]
            ).wait()
        row = base + chunk_i * _CHUNK
        pltpu.sync_copy(rows[b], out_hbm.at[pl.ds(row, _CHUNK)])

    prep(0, 0)
    prep(1, 1)

    def loop(g2, c):
        c0 = 2 * g2
        finish(c0, 0)
        prep(c0 + 2, 0)
        finish(c0 + 1, 1)
        prep(c0 + 3, 1)
        return c

    lax.fori_loop(0, _CHUNKS_PER_W // 2 - 1, loop, 0)
    finish(_CHUNKS_PER_W - 2, 0)
    finish(_CHUNKS_PER_W - 1, 1)


@jax.jit
def kernel(x, table):
    x2 = x.reshape(_NROW, _W)
    run = functools.partial(
        pl.kernel,
        mesh=plsc.VectorSubcoreMesh(core_axis_name="c", subcore_axis_name="s"),
        out_type=jax.ShapeDtypeStruct((_NROW, _W, EMB_DIM), jnp.float32),
        scratch_types=[
            pltpu.VMEM((_CHUNK, _W), jnp.int32),
            pltpu.VMEM((_CHUNK, _W), jnp.int32),
            pltpu.VMEM((_CHUNK, _W, EMB_DIM), jnp.float32),
            pltpu.VMEM((_CHUNK, _W, EMB_DIM), jnp.float32),
            pltpu.SemaphoreType.DMA,
            pltpu.SemaphoreType.DMA,
        ],
        compiler_params=pltpu.CompilerParams(use_tc_tiling_on_sc=False),
    )(_sc_body)
    out = run(x2, table)
    return out.reshape(_ROWS, _COLS, EMB_DIM)
